# baseline (speedup 1.0000x reference)
import functools

import jax
import jax.numpy as jnp
from jax import lax
from jax.experimental import pallas as pl
from jax.experimental.pallas import tpu as pltpu
from jax.experimental.pallas import tpu_sc as plsc

K = 8192
C = 32
BM = 512
BJ = 1024


def _argmin_kernel(x_ref, e_ref, idx_ref):
    x = x_ref[...]
    ones = jnp.full((BM, 1), -1.0, jnp.float32)
    xa = jnp.concatenate([x * 2.0, ones], axis=1).astype(jnp.bfloat16)
    lanes = 128
    grp = BJ // lanes

    def body(j, carry):
        best, bestg = carry
        e = e_ref[pl.ds(j * BJ, BJ), :]
        e2 = jnp.sum(e * e, axis=1, keepdims=True)
        ea = jnp.concatenate([e, e2], axis=1).astype(jnp.bfloat16)
        t = lax.dot_general(xa, ea, (((1,), (1,)), ((), ())),
                            preferred_element_type=jnp.float32)
        for k in range(grp):
            tk = lax.slice(t, (0, k * lanes), (BM, (k + 1) * lanes))
            g = j * grp + k
            upd = tk > best
            best = jnp.where(upd, tk, best)
            bestg = jnp.where(upd, g, bestg)
        return best, bestg

    best0 = jnp.full((BM, lanes), -jnp.inf, jnp.float32)
    bestg0 = jnp.zeros((BM, lanes), jnp.int32)
    best, bestg = lax.fori_loop(0, K // BJ, body, (best0, bestg0))
    maxv = jnp.max(best, axis=1, keepdims=True)
    lane = lax.broadcasted_iota(jnp.int32, (BM, lanes), 1)
    col = bestg * lanes + lane
    idx_ref[...] = jnp.min(jnp.where(best == maxv, col,
                                     jnp.int32(2**31 - 1)), axis=1)


def _compute_indices(flat, embedding):
    n = flat.shape[0]
    return pl.pallas_call(
        _argmin_kernel,
        grid=(n // BM,),
        in_specs=[
            pl.BlockSpec((BM, C), lambda r: (r, 0)),
            pl.BlockSpec((K, C), lambda r: (0, 0)),
        ],
        out_specs=pl.BlockSpec((BM,), lambda r: (r,)),
        out_shape=jax.ShapeDtypeStruct((n,), jnp.int32),
        compiler_params=pltpu.CompilerParams(
            dimension_semantics=("arbitrary",)),
    )(flat, embedding)


def _make_gather(n_rows):
    info = plsc.get_sparse_core_info()
    nw = info.num_cores * info.num_subcores
    per_w = n_rows // nw
    kc = per_w // 128
    mesh = plsc.VectorSubcoreMesh(core_axis_name="c", subcore_axis_name="s")

    @functools.partial(
        pl.kernel, mesh=mesh,
        compiler_params=pltpu.CompilerParams(use_tc_tiling_on_sc=False),
        out_type=jax.ShapeDtypeStruct((n_rows, C), jnp.float32),
        scratch_types=[
            pltpu.VMEM((kc, 128), jnp.int32),
            pltpu.VMEM((per_w, C), jnp.float32),
            pltpu.SemaphoreType.DMA,
        ],
    )
    def gather(table_hbm, idx_hbm, out_hbm, idx_v, rows_v, sem):
        wid = lax.axis_index("s") * info.num_cores + lax.axis_index("c")
        pltpu.sync_copy(idx_hbm.at[pl.ds(wid * kc, kc)], idx_v)
        copies = [
            pltpu.async_copy(table_hbm.at[idx_v.at[k]],
                             rows_v.at[pl.ds(k * 128, 128)], sem)
            for k in range(kc)
        ]
        for cp in copies:
            cp.wait()
        pltpu.sync_copy(rows_v, out_hbm.at[pl.ds(wid * per_w, per_w)])

    return gather


def _ste_kernel(x_ref, q_ref, o_ref):
    x = x_ref[...]
    q = q_ref[...]
    eye = (lax.broadcasted_iota(jnp.int32, (C, C), 0) ==
           lax.broadcasted_iota(jnp.int32, (C, C), 1)).astype(jnp.float32)
    qt = lax.dot_general(eye, q, (((1,), (1,)), ((), ())),
                         precision=lax.Precision.HIGHEST,
                         preferred_element_type=jnp.float32)
    o_ref[...] = x + (qt - x)


def _ste(inputs, q):
    B, Cc, L = inputs.shape
    return pl.pallas_call(
        _ste_kernel,
        grid=(B,),
        in_specs=[
            pl.BlockSpec((None, Cc, L), lambda b: (b, 0, 0)),
            pl.BlockSpec((None, L, Cc), lambda b: (b, 0, 0)),
        ],
        out_specs=pl.BlockSpec((None, Cc, L), lambda b: (b, 0, 0)),
        out_shape=jax.ShapeDtypeStruct((B, Cc, L), jnp.float32),
    )(inputs, q)


def kernel(inputs, embedding):
    B, Cc, L = inputs.shape
    flat = jnp.transpose(inputs, (0, 2, 1)).reshape(-1, Cc)
    idx = _compute_indices(flat, embedding)
    gather = _make_gather(B * L)
    qflat = gather(embedding, idx.reshape(-1, 128))
    quant = _ste(inputs, qflat.reshape(B, L, Cc))
    return quant, idx[:, None]


# speedup vs baseline: 1.3931x; 1.3931x over previous
import functools

import jax
import jax.numpy as jnp
from jax import lax
from jax.experimental import pallas as pl
from jax.experimental.pallas import tpu as pltpu
from jax.experimental.pallas import tpu_sc as plsc

K = 8192
C = 32
BM = 512
BJ = 4096


def _argmin_kernel(x_ref, e_ref, idx_ref):
    x = x_ref[...]
    ones = jnp.full((BM, 1), -1.0, jnp.float32)
    xa = jnp.concatenate([x * 2.0, ones], axis=1).astype(jnp.bfloat16)
    lanes = 128
    grp = BJ // lanes

    def body(j, carry):
        best, bestg = carry
        e = e_ref[pl.ds(j * BJ, BJ), :]
        e2 = jnp.sum(e * e, axis=1, keepdims=True)
        ea = jnp.concatenate([e, e2], axis=1).astype(jnp.bfloat16)
        t = lax.dot_general(xa, ea, (((1,), (1,)), ((), ())),
                            preferred_element_type=jnp.float32)
        for k in range(grp):
            tk = lax.slice(t, (0, k * lanes), (BM, (k + 1) * lanes))
            g = j * grp + k
            upd = tk > best
            best = jnp.where(upd, tk, best)
            bestg = jnp.where(upd, g, bestg)
        return best, bestg

    best0 = jnp.full((BM, lanes), -jnp.inf, jnp.float32)
    bestg0 = jnp.zeros((BM, lanes), jnp.int32)
    best, bestg = lax.fori_loop(0, K // BJ, body, (best0, bestg0))
    maxv = jnp.max(best, axis=1, keepdims=True)
    lane = lax.broadcasted_iota(jnp.int32, (BM, lanes), 1)
    col = bestg * lanes + lane
    idx_ref[...] = jnp.min(jnp.where(best == maxv, col,
                                     jnp.int32(2**31 - 1)), axis=1)


def _compute_indices(flat, embedding):
    n = flat.shape[0]
    return pl.pallas_call(
        _argmin_kernel,
        grid=(n // BM,),
        in_specs=[
            pl.BlockSpec((BM, C), lambda r: (r, 0)),
            pl.BlockSpec((K, C), lambda r: (0, 0)),
        ],
        out_specs=pl.BlockSpec((BM,), lambda r: (r,)),
        out_shape=jax.ShapeDtypeStruct((n,), jnp.int32),
        compiler_params=pltpu.CompilerParams(
            dimension_semantics=("arbitrary",)),
    )(flat, embedding)


def _make_gather(n_rows):
    info = plsc.get_sparse_core_info()
    nw = info.num_cores * info.num_subcores
    per_w = n_rows // nw
    kc = per_w // 128
    mesh = plsc.VectorSubcoreMesh(core_axis_name="c", subcore_axis_name="s")

    @functools.partial(
        pl.kernel, mesh=mesh,
        compiler_params=pltpu.CompilerParams(use_tc_tiling_on_sc=False),
        out_type=jax.ShapeDtypeStruct((n_rows, C), jnp.float32),
        scratch_types=[
            pltpu.VMEM((kc, 128), jnp.int32),
            pltpu.VMEM((per_w, C), jnp.float32),
            pltpu.SemaphoreType.DMA,
        ],
    )
    def gather(table_hbm, idx_hbm, out_hbm, idx_v, rows_v, sem):
        wid = lax.axis_index("s") * info.num_cores + lax.axis_index("c")
        pltpu.sync_copy(idx_hbm.at[pl.ds(wid * kc, kc)], idx_v)
        copies = [
            pltpu.async_copy(table_hbm.at[idx_v.at[k]],
                             rows_v.at[pl.ds(k * 128, 128)], sem)
            for k in range(kc)
        ]
        for cp in copies:
            cp.wait()
        pltpu.sync_copy(rows_v, out_hbm.at[pl.ds(wid * per_w, per_w)])

    return gather


def _ste_kernel(x_ref, q_ref, o_ref):
    x = x_ref[...]
    q = q_ref[...]
    eye = (lax.broadcasted_iota(jnp.int32, (C, C), 0) ==
           lax.broadcasted_iota(jnp.int32, (C, C), 1)).astype(jnp.float32)
    qt = lax.dot_general(eye, q, (((1,), (1,)), ((), ())),
                         precision=lax.Precision.HIGHEST,
                         preferred_element_type=jnp.float32)
    o_ref[...] = x + (qt - x)


def _ste(inputs, q):
    B, Cc, L = inputs.shape
    return pl.pallas_call(
        _ste_kernel,
        grid=(B,),
        in_specs=[
            pl.BlockSpec((None, Cc, L), lambda b: (b, 0, 0)),
            pl.BlockSpec((None, L, Cc), lambda b: (b, 0, 0)),
        ],
        out_specs=pl.BlockSpec((None, Cc, L), lambda b: (b, 0, 0)),
        out_shape=jax.ShapeDtypeStruct((B, Cc, L), jnp.float32),
    )(inputs, q)


def kernel(inputs, embedding):
    B, Cc, L = inputs.shape
    flat = jnp.transpose(inputs, (0, 2, 1)).reshape(-1, Cc)
    idx = _compute_indices(flat, embedding)
    gather = _make_gather(B * L)
    qflat = gather(embedding, idx.reshape(-1, 128))
    quant = _ste(inputs, qflat.reshape(B, L, Cc))
    return quant, idx[:, None]


# speedup vs baseline: 1.4694x; 1.0547x over previous
import functools

import jax
import jax.numpy as jnp
from jax import lax
from jax.experimental import pallas as pl
from jax.experimental.pallas import tpu as pltpu
from jax.experimental.pallas import tpu_sc as plsc

K = 8192
C = 32
BM = 512
BJ = 8192


def _argmin_kernel(x_ref, e_ref, idx_ref):
    x = x_ref[...]
    ones = jnp.full((BM, 1), -1.0, jnp.float32)
    xa = jnp.concatenate([x * 2.0, ones], axis=1).astype(jnp.bfloat16)
    lanes = 128
    grp = BJ // lanes

    def body(j, carry):
        best, bestg = carry
        e = e_ref[pl.ds(j * BJ, BJ), :]
        e2 = jnp.sum(e * e, axis=1, keepdims=True)
        ea = jnp.concatenate([e, e2], axis=1).astype(jnp.bfloat16)
        t = lax.dot_general(xa, ea, (((1,), (1,)), ((), ())),
                            preferred_element_type=jnp.float32)
        for k in range(grp):
            tk = lax.slice(t, (0, k * lanes), (BM, (k + 1) * lanes))
            g = j * grp + k
            upd = tk > best
            best = jnp.where(upd, tk, best)
            bestg = jnp.where(upd, g, bestg)
        return best, bestg

    best0 = jnp.full((BM, lanes), -jnp.inf, jnp.float32)
    bestg0 = jnp.zeros((BM, lanes), jnp.int32)
    best, bestg = lax.fori_loop(0, K // BJ, body, (best0, bestg0))
    maxv = jnp.max(best, axis=1, keepdims=True)
    lane = lax.broadcasted_iota(jnp.int32, (BM, lanes), 1)
    col = bestg * lanes + lane
    idx_ref[...] = jnp.min(jnp.where(best == maxv, col,
                                     jnp.int32(2**31 - 1)), axis=1)


def _compute_indices(flat, embedding):
    n = flat.shape[0]
    return pl.pallas_call(
        _argmin_kernel,
        grid=(n // BM,),
        in_specs=[
            pl.BlockSpec((BM, C), lambda r: (r, 0)),
            pl.BlockSpec((K, C), lambda r: (0, 0)),
        ],
        out_specs=pl.BlockSpec((BM,), lambda r: (r,)),
        out_shape=jax.ShapeDtypeStruct((n,), jnp.int32),
        compiler_params=pltpu.CompilerParams(
            dimension_semantics=("arbitrary",)),
    )(flat, embedding)


def _make_gather(n_rows):
    info = plsc.get_sparse_core_info()
    nw = info.num_cores * info.num_subcores
    per_w = n_rows // nw
    kc = per_w // 128
    mesh = plsc.VectorSubcoreMesh(core_axis_name="c", subcore_axis_name="s")

    @functools.partial(
        pl.kernel, mesh=mesh,
        compiler_params=pltpu.CompilerParams(use_tc_tiling_on_sc=False),
        out_type=jax.ShapeDtypeStruct((n_rows, C), jnp.float32),
        scratch_types=[
            pltpu.VMEM((kc, 128), jnp.int32),
            pltpu.VMEM((per_w, C), jnp.float32),
            pltpu.SemaphoreType.DMA,
        ],
    )
    def gather(table_hbm, idx_hbm, out_hbm, idx_v, rows_v, sem):
        wid = lax.axis_index("s") * info.num_cores + lax.axis_index("c")
        pltpu.sync_copy(idx_hbm.at[pl.ds(wid * kc, kc)], idx_v)
        copies = [
            pltpu.async_copy(table_hbm.at[idx_v.at[k]],
                             rows_v.at[pl.ds(k * 128, 128)], sem)
            for k in range(kc)
        ]
        for cp in copies:
            cp.wait()
        pltpu.sync_copy(rows_v, out_hbm.at[pl.ds(wid * per_w, per_w)])

    return gather


def _ste_kernel(x_ref, q_ref, o_ref):
    x = x_ref[...]
    q = q_ref[...]
    eye = (lax.broadcasted_iota(jnp.int32, (C, C), 0) ==
           lax.broadcasted_iota(jnp.int32, (C, C), 1)).astype(jnp.float32)
    qt = lax.dot_general(eye, q, (((1,), (1,)), ((), ())),
                         precision=lax.Precision.HIGHEST,
                         preferred_element_type=jnp.float32)
    o_ref[...] = x + (qt - x)


def _ste(inputs, q):
    B, Cc, L = inputs.shape
    return pl.pallas_call(
        _ste_kernel,
        grid=(B,),
        in_specs=[
            pl.BlockSpec((None, Cc, L), lambda b: (b, 0, 0)),
            pl.BlockSpec((None, L, Cc), lambda b: (b, 0, 0)),
        ],
        out_specs=pl.BlockSpec((None, Cc, L), lambda b: (b, 0, 0)),
        out_shape=jax.ShapeDtypeStruct((B, Cc, L), jnp.float32),
    )(inputs, q)


def kernel(inputs, embedding):
    B, Cc, L = inputs.shape
    flat = jnp.transpose(inputs, (0, 2, 1)).reshape(-1, Cc)
    idx = _compute_indices(flat, embedding)
    gather = _make_gather(B * L)
    qflat = gather(embedding, idx.reshape(-1, 128))
    quant = _ste(inputs, qflat.reshape(B, L, Cc))
    return quant, idx[:, None]


# speedup vs baseline: 1.4952x; 1.0176x over previous
import functools

import jax
import jax.numpy as jnp
from jax import lax
from jax.experimental import pallas as pl
from jax.experimental.pallas import tpu as pltpu
from jax.experimental.pallas import tpu_sc as plsc

K = 8192
C = 32
BM = 1024
BJ = 8192


def _argmin_kernel(x_ref, e_ref, idx_ref):
    x = x_ref[...]
    ones = jnp.full((BM, 1), -1.0, jnp.float32)
    xa = jnp.concatenate([x * 2.0, ones], axis=1).astype(jnp.bfloat16)
    lanes = 128
    grp = BJ // lanes

    def body(j, carry):
        best, bestg = carry
        e = e_ref[pl.ds(j * BJ, BJ), :]
        e2 = jnp.sum(e * e, axis=1, keepdims=True)
        ea = jnp.concatenate([e, e2], axis=1).astype(jnp.bfloat16)
        t = lax.dot_general(xa, ea, (((1,), (1,)), ((), ())),
                            preferred_element_type=jnp.float32)
        for k in range(grp):
            tk = lax.slice(t, (0, k * lanes), (BM, (k + 1) * lanes))
            g = j * grp + k
            upd = tk > best
            best = jnp.where(upd, tk, best)
            bestg = jnp.where(upd, g, bestg)
        return best, bestg

    best0 = jnp.full((BM, lanes), -jnp.inf, jnp.float32)
    bestg0 = jnp.zeros((BM, lanes), jnp.int32)
    best, bestg = lax.fori_loop(0, K // BJ, body, (best0, bestg0))
    maxv = jnp.max(best, axis=1, keepdims=True)
    lane = lax.broadcasted_iota(jnp.int32, (BM, lanes), 1)
    col = bestg * lanes + lane
    idx_ref[...] = jnp.min(jnp.where(best == maxv, col,
                                     jnp.int32(2**31 - 1)), axis=1)


def _compute_indices(flat, embedding):
    n = flat.shape[0]
    return pl.pallas_call(
        _argmin_kernel,
        grid=(n // BM,),
        in_specs=[
            pl.BlockSpec((BM, C), lambda r: (r, 0)),
            pl.BlockSpec((K, C), lambda r: (0, 0)),
        ],
        out_specs=pl.BlockSpec((BM,), lambda r: (r,)),
        out_shape=jax.ShapeDtypeStruct((n,), jnp.int32),
        compiler_params=pltpu.CompilerParams(
            dimension_semantics=("arbitrary",)),
    )(flat, embedding)


def _make_gather(n_rows):
    info = plsc.get_sparse_core_info()
    nw = info.num_cores * info.num_subcores
    per_w = n_rows // nw
    kc = per_w // 128
    mesh = plsc.VectorSubcoreMesh(core_axis_name="c", subcore_axis_name="s")

    @functools.partial(
        pl.kernel, mesh=mesh,
        compiler_params=pltpu.CompilerParams(use_tc_tiling_on_sc=False),
        out_type=jax.ShapeDtypeStruct((n_rows, C), jnp.float32),
        scratch_types=[
            pltpu.VMEM((kc, 128), jnp.int32),
            pltpu.VMEM((per_w, C), jnp.float32),
            pltpu.SemaphoreType.DMA,
        ],
    )
    def gather(table_hbm, idx_hbm, out_hbm, idx_v, rows_v, sem):
        wid = lax.axis_index("s") * info.num_cores + lax.axis_index("c")
        pltpu.sync_copy(idx_hbm.at[pl.ds(wid * kc, kc)], idx_v)
        copies = [
            pltpu.async_copy(table_hbm.at[idx_v.at[k]],
                             rows_v.at[pl.ds(k * 128, 128)], sem)
            for k in range(kc)
        ]
        for cp in copies:
            cp.wait()
        pltpu.sync_copy(rows_v, out_hbm.at[pl.ds(wid * per_w, per_w)])

    return gather


def _ste_kernel(x_ref, q_ref, o_ref):
    x = x_ref[...]
    q = q_ref[...]
    eye = (lax.broadcasted_iota(jnp.int32, (C, C), 0) ==
           lax.broadcasted_iota(jnp.int32, (C, C), 1)).astype(jnp.float32)
    qt = lax.dot_general(eye, q, (((1,), (1,)), ((), ())),
                         precision=lax.Precision.HIGHEST,
                         preferred_element_type=jnp.float32)
    o_ref[...] = x + (qt - x)


def _ste(inputs, q):
    B, Cc, L = inputs.shape
    return pl.pallas_call(
        _ste_kernel,
        grid=(B,),
        in_specs=[
            pl.BlockSpec((None, Cc, L), lambda b: (b, 0, 0)),
            pl.BlockSpec((None, L, Cc), lambda b: (b, 0, 0)),
        ],
        out_specs=pl.BlockSpec((None, Cc, L), lambda b: (b, 0, 0)),
        out_shape=jax.ShapeDtypeStruct((B, Cc, L), jnp.float32),
    )(inputs, q)


def kernel(inputs, embedding):
    B, Cc, L = inputs.shape
    flat = jnp.transpose(inputs, (0, 2, 1)).reshape(-1, Cc)
    idx = _compute_indices(flat, embedding)
    gather = _make_gather(B * L)
    qflat = gather(embedding, idx.reshape(-1, 128))
    quant = _ste(inputs, qflat.reshape(B, L, Cc))
    return quant, idx[:, None]
